# baseline (speedup 1.0000x reference)
import functools

import jax
import jax.numpy as jnp
from jax import lax
from jax.experimental import pallas as pl
from jax.experimental.pallas import tpu as pltpu
from jax.experimental.pallas import tpu_sc as plsc

BATCH = 16384
D = 64
NC = 2
NS = 16
NW = NC * NS
BPW = BATCH // NW
CHUNK = 128
NCH = BPW // CHUNK

_mesh = plsc.VectorSubcoreMesh(core_axis_name="c", subcore_axis_name="s")


@functools.partial(
    pl.kernel,
    mesh=_mesh,
    out_type=jax.ShapeDtypeStruct((BATCH,), jnp.float32),
    compiler_params=pltpu.CompilerParams(use_tc_tiling_on_sc=False),
    scratch_types=[
        pltpu.VMEM((NCH, CHUNK), jnp.int32),
        pltpu.VMEM((BPW, D), jnp.float32),
        pltpu.VMEM((BPW, D), jnp.float32),
        pltpu.VMEM((BPW, D), jnp.float32),
        pltpu.VMEM((BPW,), jnp.float32),
        pltpu.SemaphoreType.DMA,
        pltpu.SemaphoreType.DMA,
    ],
)
def _distmult_sc(head_hbm, rel_hbm, tail_hbm, emb_hbm, out_hbm,
                 idx_v, r_v, h_v, t_v, o_v, gsem, dsem):
    wid = lax.axis_index("s") * NC + lax.axis_index("c")
    base = wid * BPW

    pltpu.sync_copy(rel_hbm.at[pl.ds(wid * NCH, NCH)], idx_v)

    gathers = [
        pltpu.async_copy(emb_hbm.at[idx_v.at[k]],
                         r_v.at[pl.ds(k * CHUNK, CHUNK)], gsem)
        for k in range(NCH)
    ]
    cp_h = pltpu.async_copy(head_hbm.at[pl.ds(base, BPW)], h_v, dsem)
    cp_t = pltpu.async_copy(tail_hbm.at[pl.ds(base, BPW)], t_v, dsem)
    for cp in gathers:
        cp.wait()
    cp_h.wait()
    cp_t.wait()

    lane = lax.iota(jnp.int32, 16)

    def lane_sum(v):
        for s in (8, 4, 2, 1):
            v = v + v.at[lane ^ s].get(mode="promise_in_bounds")
        return v

    def group(g, carry):
        vec = jnp.zeros((16,), jnp.float32)
        for jj in range(16):
            j = g * 16 + jj
            acc = h_v[j, pl.ds(0, 16)] * r_v[j, pl.ds(0, 16)] * t_v[j, pl.ds(0, 16)]
            for q in range(1, D // 16):
                acc = acc + (h_v[j, pl.ds(16 * q, 16)]
                             * r_v[j, pl.ds(16 * q, 16)]
                             * t_v[j, pl.ds(16 * q, 16)])
            vec = jnp.where(lane == jj, lane_sum(acc), vec)
        o_v[pl.ds(g * 16, 16)] = vec
        return carry

    lax.fori_loop(0, BPW // 16, group, 0)

    pltpu.sync_copy(o_v, out_hbm.at[pl.ds(base, BPW)])


def kernel(head, rel, tail, rel_emb):
    rel2 = rel.astype(jnp.int32).reshape(NW * NCH, CHUNK)
    return _distmult_sc(head, rel2, tail, rel_emb)


# speedup vs baseline: 1.0974x; 1.0974x over previous
import functools

import jax
import jax.numpy as jnp
from jax import lax
from jax.experimental import pallas as pl
from jax.experimental.pallas import tpu as pltpu
from jax.experimental.pallas import tpu_sc as plsc

BATCH = 16384
D = 64
NC = 2
NS = 16
NW = NC * NS
BPW = BATCH // NW
CHUNK = 128
NCH = BPW // CHUNK
PITCH = CHUNK + 1

_mesh = plsc.VectorSubcoreMesh(core_axis_name="c", subcore_axis_name="s")


@functools.partial(
    pl.kernel,
    mesh=_mesh,
    out_type=jax.ShapeDtypeStruct((BATCH,), jnp.float32),
    compiler_params=pltpu.CompilerParams(needs_layout_passes=False),
    scratch_types=[
        pltpu.VMEM((BPW,), jnp.int32),
        pltpu.VMEM((NCH, CHUNK), jnp.int32),
        pltpu.VMEM((BPW,), jnp.int32),
        pltpu.VMEM((CHUNK, 2 * D), jnp.float32),
        pltpu.VMEM((CHUNK, 2 * D), jnp.float32),
        pltpu.VMEM((D * PITCH,), jnp.float32),
        pltpu.VMEM((D, BPW), jnp.float32),
        pltpu.VMEM((D, BPW), jnp.float32),
        pltpu.VMEM((BPW,), jnp.float32),
        pltpu.SemaphoreType.DMA,
        pltpu.SemaphoreType.DMA,
    ],
)
def _distmult_sc(ht_hbm, rel_hbm, tt_hbm, emb2_hbm, out_hbm,
                 idx_v, idx2_v, p64_v, r0_v, r1_v, rt_v, h_v, t_v, o_v,
                 gsem, dsem):
    wid = lax.axis_index("s") * NC + lax.axis_index("c")
    base = wid * BPW

    pltpu.sync_copy(rel_hbm.at[pl.ds(base, BPW)], idx_v)
    for k in range(NCH):
        for c in range(CHUNK // 16):
            iv = idx_v[pl.ds(k * CHUNK + c * 16, 16)]
            idx2_v[k, pl.ds(c * 16, 16)] = iv >> 1
            p64_v[pl.ds(k * CHUNK + c * 16, 16)] = (iv & 1) * 64

    rbufs = [r0_v, r1_v]

    def fire_gather(k):
        return pltpu.async_copy(emb2_hbm.at[idx2_v.at[k]], rbufs[k % 2], gsem)

    gathers = {0: fire_gather(0)}
    cp_h = pltpu.async_copy(ht_hbm.at[:, pl.ds(base, BPW)], h_v, dsem)
    cp_t = pltpu.async_copy(tt_hbm.at[:, pl.ds(base, BPW)], t_v, dsem)

    lane = lax.iota(jnp.int32, 16)
    scatter_addr = [(16 * q + lane) * PITCH for q in range(D // 16)]

    for k in range(NCH):
        if k + 1 < NCH:
            gathers[k + 1] = fire_gather(k + 1)
        gathers[k].wait()
        rbuf = rbufs[k % 2]

        def trans(g, carry, k=k, rbuf=rbuf):
            pv = p64_v[pl.ds(k * CHUNK + g * 16, 16)]
            for j in range(16):
                e = g * 16 + j
                p = pv[j]
                for q in range(D // 16):
                    v = rbuf[e, pl.ds(p + 16 * q, 16)]
                    plsc.store_scatter(rt_v, [scatter_addr[q] + e], v)
            return carry

        lax.fori_loop(0, CHUNK // 16, trans, 0)

        if k == 0:
            cp_h.wait()
            cp_t.wait()

        def group(g, carry, k=k):
            eoff = k * CHUNK + g * 16
            acc = jnp.zeros((16,), jnp.float32)
            for d in range(D):
                acc = acc + (rt_v[pl.ds(d * PITCH + g * 16, 16)]
                             * h_v[d, pl.ds(eoff, 16)]
                             * t_v[d, pl.ds(eoff, 16)])
            o_v[pl.ds(eoff, 16)] = acc
            return carry

        lax.fori_loop(0, CHUNK // 16, group, 0)

    pltpu.sync_copy(o_v, out_hbm.at[pl.ds(base, BPW)])


def kernel(head, rel, tail, rel_emb):
    emb2 = rel_emb.reshape(rel_emb.shape[0] // 2, 2 * D)
    return _distmult_sc(head.T, rel.astype(jnp.int32), tail.T, emb2)


# speedup vs baseline: 1.6448x; 1.4988x over previous
import functools

import jax
import jax.numpy as jnp
from jax import lax
from jax.experimental import pallas as pl
from jax.experimental.pallas import tpu as pltpu
from jax.experimental.pallas import tpu_sc as plsc

BATCH = 16384
D = 64
NREL = 100000
NC = 2
NS = 16
HALF = BATCH // NC
DPT = D // NS
CHUNK = HALF // 2
OUTW = HALF // NS

_mesh = plsc.VectorSubcoreMesh(core_axis_name="c", subcore_axis_name="s")


@functools.partial(
    pl.kernel,
    mesh=_mesh,
    out_type=jax.ShapeDtypeStruct((BATCH,), jnp.float32),
    compiler_params=pltpu.CompilerParams(needs_layout_passes=False),
    scratch_types=[
        pltpu.VMEM((NREL,), jnp.float32),
        pltpu.VMEM((HALF,), jnp.int32),
        pltpu.VMEM((CHUNK,), jnp.float32),
        pltpu.VMEM((CHUNK,), jnp.float32),
        pltpu.VMEM((HALF,), jnp.float32),
        pltpu.VMEM((OUTW,), jnp.float32),
        pltpu.VMEM_SHARED((NS, HALF // 2), jnp.float32),
        pltpu.SemaphoreType.DMA,
        pltpu.SemaphoreType.DMA,
    ],
)
def _distmult_sc(ht_hbm, rel_hbm, tt_hbm, et_hbm, out_hbm,
                 row_v, idx_v, h_v, t_v, part_v, o_v, shared_s,
                 rsem, csem):
    sc = lax.axis_index("c")
    tec = lax.axis_index("s")
    base = sc * HALF

    cp_idx = pltpu.async_copy(rel_hbm.at[pl.ds(base, HALF)], idx_v, csem)
    d0 = tec * DPT
    cp_row = pltpu.async_copy(et_hbm.at[d0, :], row_v, rsem)
    cp_idx.wait()

    for q in range(DPT):
        dd = d0 + q
        for c in range(HALF // CHUNK):
            cb = base + c * CHUNK
            cp_h = pltpu.async_copy(ht_hbm.at[dd, pl.ds(cb, CHUNK)], h_v, csem)
            cp_t = pltpu.async_copy(tt_hbm.at[dd, pl.ds(cb, CHUNK)], t_v, csem)
            if c == 0:
                cp_row.wait()
            cp_h.wait()
            cp_t.wait()

            def group(g, carry, q=q, c=c):
                eoff = c * CHUNK + g * 16
                iv = idx_v[pl.ds(eoff, 16)]
                rv = plsc.load_gather(row_v, [iv])
                v = rv * h_v[pl.ds(g * 16, 16)] * t_v[pl.ds(g * 16, 16)]
                if q == 0:
                    part_v[pl.ds(eoff, 16)] = v
                else:
                    part_v[pl.ds(eoff, 16)] = part_v[pl.ds(eoff, 16)] + v
                return carry

            lax.fori_loop(0, CHUNK // 16, group, 0)
        if q + 1 < DPT:
            cp_row = pltpu.async_copy(et_hbm.at[d0 + q + 1, :], row_v, rsem)

    ph_mine = tec // (NS // 2)
    for ph in range(2):
        pltpu.sync_copy(part_v.at[pl.ds(ph * (HALF // 2), HALF // 2)],
                        shared_s.at[tec])
        plsc.subcore_barrier()

        @pl.when(ph_mine == ph)
        def _read(ph=ph):
            for i in range(NS):
                pltpu.sync_copy(
                    shared_s.at[i, pl.ds(tec * OUTW - ph * (HALF // 2), OUTW)],
                    row_v.at[pl.ds(i * OUTW, OUTW)])

        plsc.subcore_barrier()

    def osum(g, carry):
        acc = row_v[pl.ds(g * 16, 16)]
        for i in range(1, NS):
            acc = acc + row_v[pl.ds(i * OUTW + g * 16, 16)]
        o_v[pl.ds(g * 16, 16)] = acc
        return carry

    lax.fori_loop(0, OUTW // 16, osum, 0)

    pltpu.sync_copy(o_v, out_hbm.at[pl.ds(base + tec * OUTW, OUTW)])


def kernel(head, rel, tail, rel_emb):
    return _distmult_sc(head.T, rel.astype(jnp.int32), tail.T, rel_emb.T)
